# baseline (speedup 1.0000x reference)
import functools

import jax
import jax.numpy as jnp
from jax import lax
from jax.experimental import pallas as pl
from jax.experimental.pallas import tpu as pltpu
from jax.experimental.pallas import tpu_sc as plsc

N = 100000
E = 3200000
NP = 100352
NC, NS = 2, 16
WB = 128
KG = 4
G_TOTAL = E // WB + 88
E_PAD = G_TOTAL * WB
ROWS_PER_TILE = NP // NS
RB = 1024
NBLK = NP // RB
NPP = NP // 8
RBP = RB // 8

_mesh = lambda: plsc.VectorSubcoreMesh(core_axis_name="c", subcore_axis_name="s")


@functools.cache
def _deg_count_kernel():
  @functools.partial(
      pl.kernel,
      out_type=jax.ShapeDtypeStruct((NC, NP), jnp.float32),
      mesh=_mesh(),
      compiler_params=pltpu.CompilerParams(use_tc_tiling_on_sc=False),
      scratch_types=[
          pltpu.VMEM_SHARED((NP,), jnp.float32),
          pltpu.VMEM((KG, WB), jnp.int32),
          pltpu.VMEM((WB,), jnp.float32),
      ],
  )
  def deg_count(dst2d, zeros1d, ones_w, out, acc, idx_d, ones_v):
    c = lax.axis_index("c")
    s = lax.axis_index("s")
    r0 = s * ROWS_PER_TILE
    pltpu.sync_copy(zeros1d.at[pl.ds(r0, ROWS_PER_TILE)],
                    acc.at[pl.ds(r0, ROWS_PER_TILE)])
    pltpu.sync_copy(ones_w, ones_v)
    plsc.subcore_barrier()
    g_sc = G_TOTAL // NC
    g_tile = g_sc // NS
    n_chunk = g_tile // KG
    base = c * g_sc + s * g_tile

    def body(t, carry):
      g0 = base + t * KG
      pltpu.sync_copy(dst2d.at[pl.ds(g0, KG)], idx_d)
      for j in range(KG):
        pltpu.sync_copy(ones_v, acc.at[idx_d.at[j]], add=True)
      return carry

    lax.fori_loop(0, n_chunk, body, 0)
    plsc.subcore_barrier()
    pltpu.sync_copy(acc.at[pl.ds(r0, ROWS_PER_TILE)],
                    out.at[c].at[pl.ds(r0, ROWS_PER_TILE)])

  return deg_count


@functools.cache
def _edge_pass_kernel(n_chunks):
  n_per_sc = n_chunks // NC
  g_tile = G_TOTAL // NS
  n_chunk = g_tile // KG

  @functools.partial(
      pl.kernel,
      out_type=jax.ShapeDtypeStruct((n_chunks, NP, 16), jnp.float32),
      mesh=_mesh(),
      compiler_params=pltpu.CompilerParams(use_tc_tiling_on_sc=False),
      scratch_types=[
          pltpu.VMEM_SHARED((NP, 16), jnp.float32),
          pltpu.VMEM((3, KG * WB), jnp.int32),
          pltpu.VMEM((3, KG * WB), jnp.int32),
          pltpu.VMEM((3, KG * WB, 16), jnp.float32),
          pltpu.SemaphoreType.DMA,
          pltpu.SemaphoreType.DMA,
          pltpu.SemaphoreType.DMA,
          pltpu.SemaphoreType.DMA,
          pltpu.SemaphoreType.DMA,
          pltpu.SemaphoreType.DMA,
      ],
  )
  def edge_pass(src1d, dst1d, table, zeros2d, zeros3, out,
                acc, idx_s, idx_d, rows, g0sem, g1sem, g2sem, ssem, isem,
                isem2):
    c = lax.axis_index("c")
    s = lax.axis_index("s")
    gsems = (g0sem, g1sem, g2sem)
    r0 = s * ROWS_PER_TILE
    base = s * g_tile * WB
    kb = KG * WB

    def load_idx(t, buf, sync):
      e0 = base + t * kb
      if sync:
        pltpu.sync_copy(src1d.at[pl.ds(e0, kb)], idx_s.at[buf])
        pltpu.sync_copy(dst1d.at[pl.ds(e0, kb)], idx_d.at[buf])
      else:
        pltpu.async_copy(src1d.at[pl.ds(e0, kb)], idx_s.at[buf], isem)
        pltpu.async_copy(dst1d.at[pl.ds(e0, kb)], idx_d.at[buf], isem2)

    def fire_gathers(chunk, buf):
      pltpu.async_copy(table.at[chunk].at[idx_s.at[buf]],
                       rows.at[buf], gsems[buf])

    def drain_rows(sem, buf):
      pltpu.make_async_copy(zeros3, rows.at[buf], sem).wait()

    def drain_idx():
      pltpu.make_async_copy(src1d.at[pl.ds(0, kb)], idx_s.at[0], isem).wait()
      pltpu.make_async_copy(src1d.at[pl.ds(0, kb)], idx_d.at[0], isem2).wait()

    def step(chunk, t, a, fire_g, fire_i):
      drain_rows(gsems[a], a)
      pltpu.async_copy(rows.at[a], acc.at[idx_d.at[a]], ssem, add=True)
      if fire_g:
        drain_idx()
        fire_gathers(chunk, (a + 2) % 3)
      drain_rows(ssem, a)
      if fire_i:
        load_idx(t + 3, a, sync=False)

    for ci in range(n_per_sc):
      chunk = c + NC * ci
      pltpu.sync_copy(zeros2d.at[pl.ds(r0, ROWS_PER_TILE)],
                      acc.at[pl.ds(r0, ROWS_PER_TILE)])
      plsc.subcore_barrier()

      load_idx(0, 0, sync=True)
      load_idx(1, 1, sync=True)
      fire_gathers(chunk, 0)
      fire_gathers(chunk, 1)
      load_idx(2, 2, sync=False)

      def body(t2, carry):
        t = t2 * 3
        step(chunk, t, 0, True, True)
        step(chunk, t + 1, 1, True, True)
        step(chunk, t + 2, 2, True, True)
        return carry

      n_steady = n_chunk - 5
      lax.fori_loop(0, n_steady // 3, body, 0)
      step(chunk, n_chunk - 5, 0, True, True)
      step(chunk, n_chunk - 4, 1, True, True)
      step(chunk, n_chunk - 3, 2, True, False)
      step(chunk, n_chunk - 2, 0, False, False)
      step(chunk, n_chunk - 1, 1, False, False)

      plsc.subcore_barrier()
      pltpu.sync_copy(acc.at[pl.ds(r0, ROWS_PER_TILE)],
                      out.at[chunk].at[pl.ds(r0, ROWS_PER_TILE)])
      if ci + 1 < n_per_sc:
        plsc.subcore_barrier()

  return edge_pass


def _row_mask(val):
  i = pl.program_id(0)
  rows = i * RB + lax.broadcasted_iota(jnp.int32, (RB, 1), 0)
  return jnp.where(rows < N, val, 0.0)


def _chunk_cat(ref, n):
  return jnp.concatenate([ref[k] for k in range(n)], axis=-1)


def _jmask(j, val):
  i = pl.program_id(0)
  rows = i * RB + j * 128 + lax.broadcasted_iota(jnp.int32, (128, 1), 0)
  return jnp.where(rows < N, val, 0.0)


@functools.cache
def _tc0_kernel():
  def body(dp_ref, xt_ref, d16_ref, g_ref):
    deg = 1.0 + dp_ref[0, 0] + dp_ref[1, 0]
    dd8 = lax.rsqrt(deg).T
    d16_cols = []
    g_cols = [[], []]
    for j in range(8):
      d16_j = jnp.broadcast_to(dd8[:, j:j + 1], (128, 16))
      d16_cols.append(d16_j)
      xj = xt_ref[:, 128 * j:128 * (j + 1)].T
      x32 = jnp.concatenate([xj, jnp.zeros((128, 11), jnp.float32)], axis=1)
      gj = _jmask(j, jnp.concatenate([d16_j, d16_j], axis=1) * x32)
      g_cols[0].append(gj[:, :16])
      g_cols[1].append(gj[:, 16:])
    d16_ref[...] = jnp.concatenate(d16_cols, axis=1)
    g_ref[0] = jnp.concatenate(g_cols[0], axis=1)
    g_ref[1] = jnp.concatenate(g_cols[1], axis=1)

  return pl.pallas_call(
      body,
      grid=(NBLK,),
      in_specs=[
          pl.BlockSpec((2, 1, 8, 128), lambda i: (0, i, 0, 0)),
          pl.BlockSpec((21, RB), lambda i: (0, i)),
      ],
      out_specs=[
          pl.BlockSpec((RBP, 128), lambda i: (i, 0)),
          pl.BlockSpec((2, RBP, 128), lambda i: (0, i, 0)),
      ],
      out_shape=[
          jax.ShapeDtypeStruct((NPP, 128), jnp.float32),
          jax.ShapeDtypeStruct((2, NPP, 128), jnp.float32),
      ],
  )


@functools.cache
def _tc_join_kernel(nc_in, nc_out, cin, cout, two_mats):
  def body(acc_ref, g_ref, d_ref, w_ref, b_ref, *rest):
    if two_mats:
      w2_ref, out_ref = rest
    else:
      (out_ref,) = rest
    d16p = d_ref[...]
    pk = d16p * (acc_ref[...] + g_ref[...])
    out_cols = [[] for _ in range(nc_out)]
    for j in range(8):
      sl = slice(16 * j, 16 * (j + 1))
      p_j = jnp.concatenate([pk[k, :, sl] for k in range(nc_in)], axis=-1)
      x_j = jnp.maximum(
          jnp.dot(p_j, w_ref[...], preferred_element_type=jnp.float32)
          + b_ref[...], 0.0)
      if two_mats:
        x_j = jnp.dot(x_j, w2_ref[...], preferred_element_type=jnp.float32)
      x_j = _jmask(j, x_j)
      d_j = d16p[:, sl]
      for m in range(nc_out):
        out_cols[m].append(d_j * x_j[:, 16 * m:16 * (m + 1)])
    for m in range(nc_out):
      out_ref[m] = jnp.concatenate(out_cols[m], axis=1)

  in_specs = [
      pl.BlockSpec((nc_in, RBP, 128), lambda i: (0, i, 0)),
      pl.BlockSpec((nc_in, RBP, 128), lambda i: (0, i, 0)),
      pl.BlockSpec((RBP, 128), lambda i: (i, 0)),
      pl.BlockSpec((cin, cout), lambda i: (0, 0)),
      pl.BlockSpec((1, cout), lambda i: (0, 0)),
  ]
  if two_mats:
    in_specs.append(pl.BlockSpec((cout, 16 * nc_out), lambda i: (0, 0)))
  return pl.pallas_call(
      body,
      grid=(NBLK,),
      in_specs=in_specs,
      out_specs=pl.BlockSpec((nc_out, RBP, 128), lambda i: (0, i, 0)),
      out_shape=jax.ShapeDtypeStruct((nc_out, NPP, 128), jnp.float32),
  )


@functools.cache
def _tc_final_kernel():
  def body(acc_ref, g_ref, d_ref, b_ref, out_ref):
    d16p = d_ref[...]
    pk = d16p * (acc_ref[...] + g_ref[...])
    cols = []
    for j in range(8):
      sl = slice(16 * j, 16 * (j + 1))
      p_j = jnp.concatenate([pk[k, :, sl] for k in range(2)], axis=-1)
      res_j = p_j + b_ref[...]
      cols.append(res_j.T[:21, :])
    out_ref[...] = jnp.concatenate(cols, axis=1)

  return pl.pallas_call(
      body,
      grid=(NBLK,),
      in_specs=[
          pl.BlockSpec((2, RBP, 128), lambda i: (0, i, 0)),
          pl.BlockSpec((2, RBP, 128), lambda i: (0, i, 0)),
          pl.BlockSpec((RBP, 128), lambda i: (i, 0)),
          pl.BlockSpec((1, 32), lambda i: (0, 0)),
      ],
      out_specs=pl.BlockSpec((21, RB), lambda i: (0, i)),
      out_shape=jax.ShapeDtypeStruct((21, N), jnp.float32),
  )


def kernel(x, edge_index, W1, b1, W2, b2, W3, b3, W4, b4):
  f32 = jnp.float32
  src = edge_index[0]
  dst = edge_index[1]
  pad_idx = N + (jnp.arange(E_PAD - E, dtype=jnp.int32) % WB)
  src2d = jnp.concatenate([src, pad_idx]).reshape(G_TOTAL, WB)
  dst2d = jnp.concatenate([dst, pad_idx]).reshape(G_TOTAL, WB)

  def vperm(n):
    return (n & ~1023) | ((n & 127) << 3) | ((n >> 7) & 7)

  src2v = vperm(src2d).reshape(E_PAD)
  dst2v = vperm(dst2d).reshape(E_PAD)
  zeros1d = jnp.zeros((NP,), f32)
  zeros2d = jnp.zeros((NP, 16), f32)
  zeros3 = jnp.zeros((KG * WB, 16), f32)
  ones_w = jnp.ones((WB,), f32)

  xt = x.T
  W1p = jnp.zeros((32, 32), f32).at[:21, :].set(W1)
  b1p = b1.reshape(1, 32)
  b2p = b2.reshape(1, 64)
  b3p = b3.reshape(1, 128)
  W4p = jnp.zeros((128, 32), f32).at[:, :21].set(W4)
  b4p = jnp.zeros((1, 32), f32).at[0, :21].set(b4)

  dp = _deg_count_kernel()(dst2d, zeros1d, ones_w)
  dp4 = dp.reshape(2, NBLK, 8, 128)
  d, g1 = _tc0_kernel()(dp4, xt)

  ep2 = _edge_pass_kernel(2)
  ep4 = _edge_pass_kernel(4)

  def sc_view(a):
    return a.reshape(a.shape[0], NP, 16)

  def tc_view(a):
    return a.reshape(a.shape[0], NPP, 128)

  acc1 = tc_view(ep2(src2v, dst2v, sc_view(g1), zeros2d, zeros3))
  g2 = _tc_join_kernel(2, 2, 32, 32, False)(acc1, g1, d, W1p, b1p)
  acc2 = tc_view(ep2(src2v, dst2v, sc_view(g2), zeros2d, zeros3))
  g3 = _tc_join_kernel(2, 4, 32, 64, False)(acc2, g2, d, W2, b2p)
  acc3 = tc_view(ep4(src2v, dst2v, sc_view(g3), zeros2d, zeros3))
  g4 = _tc_join_kernel(4, 2, 64, 128, True)(acc3, g3, d, W3, b3p, W4p)
  acc4 = tc_view(ep2(src2v, dst2v, sc_view(g4), zeros2d, zeros3))
  outt = _tc_final_kernel()(acc4, g4, d, b4p)
  return outt.T


# speedup vs baseline: 1.0368x; 1.0368x over previous
import functools

import jax
import jax.numpy as jnp
from jax import lax
from jax.experimental import pallas as pl
from jax.experimental.pallas import tpu as pltpu
from jax.experimental.pallas import tpu_sc as plsc

N = 100000
E = 3200000
NP = 100352
NC, NS = 2, 16
WB = 128
KG = 4
G_TOTAL = E // WB + 88
E_PAD = G_TOTAL * WB
ROWS_PER_TILE = NP // NS
RB = 1024
NBLK = NP // RB
NPP = NP // 8
RBP = RB // 8

_mesh = lambda: plsc.VectorSubcoreMesh(core_axis_name="c", subcore_axis_name="s")


@functools.cache
def _deg_count_kernel():
  @functools.partial(
      pl.kernel,
      out_type=jax.ShapeDtypeStruct((NC, NP), jnp.float32),
      mesh=_mesh(),
      compiler_params=pltpu.CompilerParams(use_tc_tiling_on_sc=False),
      scratch_types=[
          pltpu.VMEM_SHARED((NP,), jnp.float32),
          pltpu.VMEM((KG, WB), jnp.int32),
          pltpu.VMEM((WB,), jnp.float32),
      ],
  )
  def deg_count(dst2d, zeros1d, ones_w, out, acc, idx_d, ones_v):
    c = lax.axis_index("c")
    s = lax.axis_index("s")
    r0 = s * ROWS_PER_TILE
    pltpu.sync_copy(zeros1d.at[pl.ds(r0, ROWS_PER_TILE)],
                    acc.at[pl.ds(r0, ROWS_PER_TILE)])
    pltpu.sync_copy(ones_w, ones_v)
    plsc.subcore_barrier()
    g_sc = G_TOTAL // NC
    g_tile = g_sc // NS
    n_chunk = g_tile // KG
    base = c * g_sc + s * g_tile

    def body(t, carry):
      g0 = base + t * KG
      pltpu.sync_copy(dst2d.at[pl.ds(g0, KG)], idx_d)
      for j in range(KG):
        pltpu.sync_copy(ones_v, acc.at[idx_d.at[j]], add=True)
      return carry

    lax.fori_loop(0, n_chunk, body, 0)
    plsc.subcore_barrier()
    pltpu.sync_copy(acc.at[pl.ds(r0, ROWS_PER_TILE)],
                    out.at[c].at[pl.ds(r0, ROWS_PER_TILE)])

  return deg_count


@functools.cache
def _edge_pass_kernel(n_chunks):
  n_per_sc = n_chunks // NC
  g_tile = G_TOTAL // NS
  n_chunk = g_tile // KG

  @functools.partial(
      pl.kernel,
      out_type=jax.ShapeDtypeStruct((n_chunks, NP, 16), jnp.float32),
      mesh=_mesh(),
      compiler_params=pltpu.CompilerParams(use_tc_tiling_on_sc=False),
      scratch_types=[
          pltpu.VMEM_SHARED((NP, 16), jnp.float32),
          pltpu.VMEM((3, KG * WB), jnp.int32),
          pltpu.VMEM((3, KG * WB), jnp.int32),
          pltpu.VMEM((3, KG * WB, 16), jnp.float32),
          pltpu.SemaphoreType.DMA,
          pltpu.SemaphoreType.DMA,
          pltpu.SemaphoreType.DMA,
          pltpu.SemaphoreType.DMA,
          pltpu.SemaphoreType.DMA,
          pltpu.SemaphoreType.DMA,
      ],
  )
  def edge_pass(src1d, dst1d, table, zeros2d, zeros3, out,
                acc, idx_s, idx_d, rows, g0sem, g1sem, g2sem, ssem, isem,
                isem2):
    c = lax.axis_index("c")
    s = lax.axis_index("s")
    gsems = (g0sem, g1sem, g2sem)
    r0 = s * ROWS_PER_TILE
    base = s * g_tile * WB
    kb = KG * WB

    def load_idx(t, buf, sync):
      e0 = base + t * kb
      if sync:
        pltpu.sync_copy(src1d.at[pl.ds(e0, kb)], idx_s.at[buf])
        pltpu.sync_copy(dst1d.at[pl.ds(e0, kb)], idx_d.at[buf])
      else:
        pltpu.async_copy(src1d.at[pl.ds(e0, kb)], idx_s.at[buf], isem)
        pltpu.async_copy(dst1d.at[pl.ds(e0, kb)], idx_d.at[buf], isem2)

    def fire_gathers(chunk, buf):
      pltpu.async_copy(table.at[chunk].at[idx_s.at[buf]],
                       rows.at[buf], gsems[buf])

    def drain_rows(sem, buf):
      pltpu.make_async_copy(zeros3, rows.at[buf], sem).wait()

    def drain_idx():
      pltpu.make_async_copy(src1d.at[pl.ds(0, kb)], idx_s.at[0], isem).wait()
      pltpu.make_async_copy(src1d.at[pl.ds(0, kb)], idx_d.at[0], isem2).wait()

    def step(chunk, t, a, fire_g, fire_i):
      drain_rows(gsems[a], a)
      pltpu.async_copy(rows.at[a], acc.at[idx_d.at[a]], ssem, add=True)
      if fire_g:
        drain_idx()
        fire_gathers(chunk, (a + 2) % 3)
      drain_rows(ssem, a)
      if fire_i:
        load_idx(t + 3, a, sync=False)

    for ci in range(n_per_sc):
      chunk = c + NC * ci
      pltpu.sync_copy(zeros2d.at[pl.ds(r0, ROWS_PER_TILE)],
                      acc.at[pl.ds(r0, ROWS_PER_TILE)])
      plsc.subcore_barrier()

      load_idx(0, 0, sync=True)
      load_idx(1, 1, sync=True)
      fire_gathers(chunk, 0)
      fire_gathers(chunk, 1)
      load_idx(2, 2, sync=False)

      def body(t2, carry):
        t = t2 * 3
        step(chunk, t, 0, True, True)
        step(chunk, t + 1, 1, True, True)
        step(chunk, t + 2, 2, True, True)
        return carry

      n_steady = n_chunk - 5
      lax.fori_loop(0, n_steady // 3, body, 0)
      step(chunk, n_chunk - 5, 0, True, True)
      step(chunk, n_chunk - 4, 1, True, True)
      step(chunk, n_chunk - 3, 2, True, False)
      step(chunk, n_chunk - 2, 0, False, False)
      step(chunk, n_chunk - 1, 1, False, False)

      plsc.subcore_barrier()
      pltpu.sync_copy(acc.at[pl.ds(r0, ROWS_PER_TILE)],
                      out.at[chunk].at[pl.ds(r0, ROWS_PER_TILE)])
      if ci + 1 < n_per_sc:
        plsc.subcore_barrier()

  return edge_pass


def _row_mask(val):
  i = pl.program_id(0)
  rows = i * RB + lax.broadcasted_iota(jnp.int32, (RB, 1), 0)
  return jnp.where(rows < N, val, 0.0)


def _chunk_cat(ref, n):
  return jnp.concatenate([ref[k] for k in range(n)], axis=-1)


def _jmask(j, val):
  i = pl.program_id(0)
  rows = i * RB + j * 128 + lax.broadcasted_iota(jnp.int32, (128, 1), 0)
  return jnp.where(rows < N, val, 0.0)


@functools.cache
def _tc0_kernel():
  def body(dp_ref, xt_ref, d16_ref, g_ref):
    deg = 1.0 + dp_ref[0, 0] + dp_ref[1, 0]
    dd8 = lax.rsqrt(deg).T
    d16_cols = []
    g_cols = [[], []]
    for j in range(8):
      d16_j = jnp.broadcast_to(dd8[:, j:j + 1], (128, 16))
      d16_cols.append(d16_j)
      xj = xt_ref[:, 128 * j:128 * (j + 1)].T
      x32 = jnp.concatenate([xj, jnp.zeros((128, 11), jnp.float32)], axis=1)
      gj = _jmask(j, jnp.concatenate([d16_j, d16_j], axis=1) * x32)
      g_cols[0].append(gj[:, :16])
      g_cols[1].append(gj[:, 16:])
    d16_ref[...] = jnp.concatenate(d16_cols, axis=1)
    g_ref[0] = jnp.concatenate(g_cols[0], axis=1)
    g_ref[1] = jnp.concatenate(g_cols[1], axis=1)

  return pl.pallas_call(
      body,
      grid=(NBLK,),
      in_specs=[
          pl.BlockSpec((2, 1, 8, 128), lambda i: (0, i, 0, 0)),
          pl.BlockSpec((21, RB), lambda i: (0, i)),
      ],
      out_specs=[
          pl.BlockSpec((RBP, 128), lambda i: (i, 0)),
          pl.BlockSpec((2, RBP, 128), lambda i: (0, i, 0)),
      ],
      out_shape=[
          jax.ShapeDtypeStruct((NPP, 128), jnp.float32),
          jax.ShapeDtypeStruct((2, NPP, 128), jnp.float32),
      ],
  )


@functools.cache
def _tc_join_kernel(nc_in, nc_out, cin, cout, two_mats):
  def body(acc_ref, g_ref, d_ref, w_ref, b_ref, *rest):
    if two_mats:
      w2_ref, out_ref = rest
    else:
      (out_ref,) = rest
    d16p = d_ref[...]
    pk = d16p * (acc_ref[...] + g_ref[...])
    p = jnp.concatenate(
        [jnp.concatenate([pk[k, :, 16 * j:16 * (j + 1)] for k in range(nc_in)],
                         axis=-1) for j in range(8)], axis=0)
    x = jnp.maximum(
        jnp.dot(p, w_ref[...], preferred_element_type=jnp.float32)
        + b_ref[...], 0.0)
    if two_mats:
      x = jnp.dot(x, w2_ref[...], preferred_element_type=jnp.float32)
    x = _row_mask(x)
    for m in range(nc_out):
      out_ref[m] = d16p * jnp.concatenate(
          [x[128 * j:128 * (j + 1), 16 * m:16 * (m + 1)] for j in range(8)],
          axis=1)

  in_specs = [
      pl.BlockSpec((nc_in, RBP, 128), lambda i: (0, i, 0)),
      pl.BlockSpec((nc_in, RBP, 128), lambda i: (0, i, 0)),
      pl.BlockSpec((RBP, 128), lambda i: (i, 0)),
      pl.BlockSpec((cin, cout), lambda i: (0, 0)),
      pl.BlockSpec((1, cout), lambda i: (0, 0)),
  ]
  if two_mats:
    in_specs.append(pl.BlockSpec((cout, 16 * nc_out), lambda i: (0, 0)))
  return pl.pallas_call(
      body,
      grid=(NBLK,),
      in_specs=in_specs,
      out_specs=pl.BlockSpec((nc_out, RBP, 128), lambda i: (0, i, 0)),
      out_shape=jax.ShapeDtypeStruct((nc_out, NPP, 128), jnp.float32),
  )


@functools.cache
def _tc_final_kernel():
  def body(acc_ref, g_ref, d_ref, b_ref, out_ref):
    d16p = d_ref[...]
    pk = d16p * (acc_ref[...] + g_ref[...])
    cols = []
    for j in range(8):
      sl = slice(16 * j, 16 * (j + 1))
      p_j = jnp.concatenate([pk[k, :, sl] for k in range(2)], axis=-1)
      res_j = p_j + b_ref[...]
      cols.append(res_j.T[:21, :])
    out_ref[...] = jnp.concatenate(cols, axis=1)

  return pl.pallas_call(
      body,
      grid=(NBLK,),
      in_specs=[
          pl.BlockSpec((2, RBP, 128), lambda i: (0, i, 0)),
          pl.BlockSpec((2, RBP, 128), lambda i: (0, i, 0)),
          pl.BlockSpec((RBP, 128), lambda i: (i, 0)),
          pl.BlockSpec((1, 32), lambda i: (0, 0)),
      ],
      out_specs=pl.BlockSpec((21, RB), lambda i: (0, i)),
      out_shape=jax.ShapeDtypeStruct((21, N), jnp.float32),
  )


def kernel(x, edge_index, W1, b1, W2, b2, W3, b3, W4, b4):
  f32 = jnp.float32
  src = edge_index[0]
  dst = edge_index[1]
  pad_idx = N + (jnp.arange(E_PAD - E, dtype=jnp.int32) % WB)
  src2d = jnp.concatenate([src, pad_idx]).reshape(G_TOTAL, WB)
  dst2d = jnp.concatenate([dst, pad_idx]).reshape(G_TOTAL, WB)

  def vperm(n):
    return (n & ~1023) | ((n & 127) << 3) | ((n >> 7) & 7)

  src2v = vperm(src2d).reshape(E_PAD)
  dst2v = vperm(dst2d).reshape(E_PAD)
  zeros1d = jnp.zeros((NP,), f32)
  zeros2d = jnp.zeros((NP, 16), f32)
  zeros3 = jnp.zeros((KG * WB, 16), f32)
  ones_w = jnp.ones((WB,), f32)

  xt = x.T
  W1p = jnp.zeros((32, 32), f32).at[:21, :].set(W1)
  b1p = b1.reshape(1, 32)
  b2p = b2.reshape(1, 64)
  b3p = b3.reshape(1, 128)
  W4p = jnp.zeros((128, 32), f32).at[:, :21].set(W4)
  b4p = jnp.zeros((1, 32), f32).at[0, :21].set(b4)

  dp = _deg_count_kernel()(dst2d, zeros1d, ones_w)
  dp4 = dp.reshape(2, NBLK, 8, 128)
  d, g1 = _tc0_kernel()(dp4, xt)

  ep2 = _edge_pass_kernel(2)
  ep4 = _edge_pass_kernel(4)

  def sc_view(a):
    return a.reshape(a.shape[0], NP, 16)

  def tc_view(a):
    return a.reshape(a.shape[0], NPP, 128)

  acc1 = tc_view(ep2(src2v, dst2v, sc_view(g1), zeros2d, zeros3))
  g2 = _tc_join_kernel(2, 2, 32, 32, False)(acc1, g1, d, W1p, b1p)
  acc2 = tc_view(ep2(src2v, dst2v, sc_view(g2), zeros2d, zeros3))
  g3 = _tc_join_kernel(2, 4, 32, 64, False)(acc2, g2, d, W2, b2p)
  acc3 = tc_view(ep4(src2v, dst2v, sc_view(g3), zeros2d, zeros3))
  g4 = _tc_join_kernel(4, 2, 64, 128, True)(acc3, g3, d, W3, b3p, W4p)
  acc4 = tc_view(ep2(src2v, dst2v, sc_view(g4), zeros2d, zeros3))
  outt = _tc_final_kernel()(acc4, g4, d, b4p)
  return outt.T


# speedup vs baseline: 1.1074x; 1.0681x over previous
import functools

import jax
import jax.numpy as jnp
from jax import lax
from jax.experimental import pallas as pl
from jax.experimental.pallas import tpu as pltpu
from jax.experimental.pallas import tpu_sc as plsc

N = 100000
E = 3200000
NP = 100352
NC, NS = 2, 16
WB = 128
KG = 4
G_TOTAL = E // WB + 88
E_PAD = G_TOTAL * WB
ROWS_PER_TILE = NP // NS
RB = 1024
NBLK = NP // RB
NPP = NP // 8
RBP = RB // 8

_mesh = lambda: plsc.VectorSubcoreMesh(core_axis_name="c", subcore_axis_name="s")


KGD = 28
KBD = KGD * WB


@functools.cache
def _deg_count_kernel():
  @functools.partial(
      pl.kernel,
      out_type=jax.ShapeDtypeStruct((NC, NP), jnp.float32),
      mesh=_mesh(),
      compiler_params=pltpu.CompilerParams(use_tc_tiling_on_sc=False),
      scratch_types=[
          pltpu.VMEM_SHARED((NP,), jnp.float32),
          pltpu.VMEM((2, KBD), jnp.int32),
          pltpu.VMEM((KBD,), jnp.float32),
          pltpu.SemaphoreType.DMA,
          pltpu.SemaphoreType.DMA,
          pltpu.SemaphoreType.DMA,
      ],
  )
  def deg_count(dst1d, zeros1d, ones_w, out, acc, idx_d, ones_v,
                s0sem, s1sem, isem):
    c = lax.axis_index("c")
    s = lax.axis_index("s")
    ssems = (s0sem, s1sem)
    r0 = s * ROWS_PER_TILE
    pltpu.sync_copy(zeros1d.at[pl.ds(r0, ROWS_PER_TILE)],
                    acc.at[pl.ds(r0, ROWS_PER_TILE)])
    pltpu.sync_copy(ones_w, ones_v)
    plsc.subcore_barrier()
    e_sc = E_PAD // NC
    e_tile = e_sc // NS
    n_chunk = e_tile // KBD
    base = c * e_sc + s * e_tile

    def load_idx(t, buf, sync):
      e0 = base + t * KBD
      if sync:
        pltpu.sync_copy(dst1d.at[pl.ds(e0, KBD)], idx_d.at[buf])
      else:
        pltpu.async_copy(dst1d.at[pl.ds(e0, KBD)], idx_d.at[buf], isem)

    def step(t, buf, first, last):
      if not first:
        pltpu.make_async_copy(zeros1d.at[pl.ds(0, KBD)],
                              idx_d.at[buf], isem).wait()
      pltpu.async_copy(ones_v, acc.at[idx_d.at[buf]], ssems[buf], add=True)
      if not first:
        pltpu.make_async_copy(zeros1d.at[pl.ds(0, KBD)],
                              ones_v, ssems[1 - buf]).wait()
      if not last:
        load_idx(t + 1, 1 - buf, sync=False)

    load_idx(0, 0, sync=True)
    step(0, 0, True, False)

    def body(t2, carry):
      t = 1 + t2 * 2
      step(t, 1, False, False)
      step(t + 1, 0, False, False)
      return carry

    lax.fori_loop(0, (n_chunk - 2) // 2, body, 0)
    step(n_chunk - 1, 1, False, True)
    pltpu.make_async_copy(zeros1d.at[pl.ds(0, KBD)],
                          ones_v, ssems[1]).wait()
    plsc.subcore_barrier()
    pltpu.sync_copy(acc.at[pl.ds(r0, ROWS_PER_TILE)],
                    out.at[c].at[pl.ds(r0, ROWS_PER_TILE)])

  return deg_count


@functools.cache
def _edge_pass_kernel(n_chunks):
  n_per_sc = n_chunks // NC
  g_tile = G_TOTAL // NS
  n_chunk = g_tile // KG

  @functools.partial(
      pl.kernel,
      out_type=jax.ShapeDtypeStruct((n_chunks, NP, 16), jnp.float32),
      mesh=_mesh(),
      compiler_params=pltpu.CompilerParams(use_tc_tiling_on_sc=False),
      scratch_types=[
          pltpu.VMEM_SHARED((NP, 16), jnp.float32),
          pltpu.VMEM((3, KG * WB), jnp.int32),
          pltpu.VMEM((3, KG * WB), jnp.int32),
          pltpu.VMEM((3, KG * WB, 16), jnp.float32),
          pltpu.SemaphoreType.DMA,
          pltpu.SemaphoreType.DMA,
          pltpu.SemaphoreType.DMA,
          pltpu.SemaphoreType.DMA,
          pltpu.SemaphoreType.DMA,
          pltpu.SemaphoreType.DMA,
      ],
  )
  def edge_pass(src1d, dst1d, table, zeros2d, zeros3, out,
                acc, idx_s, idx_d, rows, g0sem, g1sem, g2sem, ssem, isem,
                isem2):
    c = lax.axis_index("c")
    s = lax.axis_index("s")
    gsems = (g0sem, g1sem, g2sem)
    r0 = s * ROWS_PER_TILE
    base = s * g_tile * WB
    kb = KG * WB

    def load_idx(t, buf, sync):
      e0 = base + t * kb
      if sync:
        pltpu.sync_copy(src1d.at[pl.ds(e0, kb)], idx_s.at[buf])
        pltpu.sync_copy(dst1d.at[pl.ds(e0, kb)], idx_d.at[buf])
      else:
        pltpu.async_copy(src1d.at[pl.ds(e0, kb)], idx_s.at[buf], isem)
        pltpu.async_copy(dst1d.at[pl.ds(e0, kb)], idx_d.at[buf], isem2)

    def fire_gathers(chunk, buf):
      pltpu.async_copy(table.at[chunk].at[idx_s.at[buf]],
                       rows.at[buf], gsems[buf])

    def drain_rows(sem, buf):
      pltpu.make_async_copy(zeros3, rows.at[buf], sem).wait()

    def drain_idx():
      pltpu.make_async_copy(src1d.at[pl.ds(0, kb)], idx_s.at[0], isem).wait()
      pltpu.make_async_copy(src1d.at[pl.ds(0, kb)], idx_d.at[0], isem2).wait()

    def step(chunk, t, a, fire_g, fire_i):
      drain_rows(gsems[a], a)
      pltpu.async_copy(rows.at[a], acc.at[idx_d.at[a]], ssem, add=True)
      if fire_g:
        drain_idx()
        fire_gathers(chunk, (a + 2) % 3)
      drain_rows(ssem, a)
      if fire_i:
        load_idx(t + 3, a, sync=False)

    for ci in range(n_per_sc):
      chunk = c + NC * ci
      pltpu.sync_copy(zeros2d.at[pl.ds(r0, ROWS_PER_TILE)],
                      acc.at[pl.ds(r0, ROWS_PER_TILE)])
      plsc.subcore_barrier()

      load_idx(0, 0, sync=True)
      load_idx(1, 1, sync=True)
      fire_gathers(chunk, 0)
      fire_gathers(chunk, 1)
      load_idx(2, 2, sync=False)

      def body(t2, carry):
        t = t2 * 3
        step(chunk, t, 0, True, True)
        step(chunk, t + 1, 1, True, True)
        step(chunk, t + 2, 2, True, True)
        return carry

      n_steady = n_chunk - 5
      lax.fori_loop(0, n_steady // 3, body, 0)
      step(chunk, n_chunk - 5, 0, True, True)
      step(chunk, n_chunk - 4, 1, True, True)
      step(chunk, n_chunk - 3, 2, True, False)
      step(chunk, n_chunk - 2, 0, False, False)
      step(chunk, n_chunk - 1, 1, False, False)

      plsc.subcore_barrier()
      pltpu.sync_copy(acc.at[pl.ds(r0, ROWS_PER_TILE)],
                      out.at[chunk].at[pl.ds(r0, ROWS_PER_TILE)])
      if ci + 1 < n_per_sc:
        plsc.subcore_barrier()

  return edge_pass


def _row_mask(val):
  i = pl.program_id(0)
  rows = i * RB + lax.broadcasted_iota(jnp.int32, (RB, 1), 0)
  return jnp.where(rows < N, val, 0.0)


def _chunk_cat(ref, n):
  return jnp.concatenate([ref[k] for k in range(n)], axis=-1)


def _jmask(j, val):
  i = pl.program_id(0)
  rows = i * RB + j * 128 + lax.broadcasted_iota(jnp.int32, (128, 1), 0)
  return jnp.where(rows < N, val, 0.0)


@functools.cache
def _tc0_kernel():
  def body(dp_ref, xt_ref, d16_ref, g_ref):
    deg = 1.0 + dp_ref[0, 0] + dp_ref[1, 0]
    dd8 = lax.rsqrt(deg).T
    d16_cols = []
    g_cols = [[], []]
    for j in range(8):
      d16_j = jnp.broadcast_to(dd8[:, j:j + 1], (128, 16))
      d16_cols.append(d16_j)
      xj = xt_ref[:, 128 * j:128 * (j + 1)].T
      x32 = jnp.concatenate([xj, jnp.zeros((128, 11), jnp.float32)], axis=1)
      gj = _jmask(j, jnp.concatenate([d16_j, d16_j], axis=1) * x32)
      g_cols[0].append(gj[:, :16])
      g_cols[1].append(gj[:, 16:])
    d16_ref[...] = jnp.concatenate(d16_cols, axis=1)
    g_ref[0] = jnp.concatenate(g_cols[0], axis=1)
    g_ref[1] = jnp.concatenate(g_cols[1], axis=1)

  return pl.pallas_call(
      body,
      grid=(NBLK,),
      in_specs=[
          pl.BlockSpec((2, 1, 8, 128), lambda i: (0, i, 0, 0)),
          pl.BlockSpec((21, RB), lambda i: (0, i)),
      ],
      out_specs=[
          pl.BlockSpec((RBP, 128), lambda i: (i, 0)),
          pl.BlockSpec((2, RBP, 128), lambda i: (0, i, 0)),
      ],
      out_shape=[
          jax.ShapeDtypeStruct((NPP, 128), jnp.float32),
          jax.ShapeDtypeStruct((2, NPP, 128), jnp.float32),
      ],
  )


@functools.cache
def _tc_join_kernel(nc_in, nc_out, cin, cout, two_mats):
  def body(acc_ref, g_ref, d_ref, w_ref, b_ref, *rest):
    if two_mats:
      w2_ref, out_ref = rest
    else:
      (out_ref,) = rest
    d16p = d_ref[...]
    pk = d16p * (acc_ref[...] + g_ref[...])
    p = jnp.concatenate(
        [jnp.concatenate([pk[k, :, 16 * j:16 * (j + 1)] for k in range(nc_in)],
                         axis=-1) for j in range(8)], axis=0)
    x = jnp.maximum(
        jnp.dot(p, w_ref[...], preferred_element_type=jnp.float32)
        + b_ref[...], 0.0)
    if two_mats:
      x = jnp.dot(x, w2_ref[...], preferred_element_type=jnp.float32)
    x = _row_mask(x)
    for m in range(nc_out):
      out_ref[m] = d16p * jnp.concatenate(
          [x[128 * j:128 * (j + 1), 16 * m:16 * (m + 1)] for j in range(8)],
          axis=1)

  in_specs = [
      pl.BlockSpec((nc_in, RBP, 128), lambda i: (0, i, 0)),
      pl.BlockSpec((nc_in, RBP, 128), lambda i: (0, i, 0)),
      pl.BlockSpec((RBP, 128), lambda i: (i, 0)),
      pl.BlockSpec((cin, cout), lambda i: (0, 0)),
      pl.BlockSpec((1, cout), lambda i: (0, 0)),
  ]
  if two_mats:
    in_specs.append(pl.BlockSpec((cout, 16 * nc_out), lambda i: (0, 0)))
  return pl.pallas_call(
      body,
      grid=(NBLK,),
      in_specs=in_specs,
      out_specs=pl.BlockSpec((nc_out, RBP, 128), lambda i: (0, i, 0)),
      out_shape=jax.ShapeDtypeStruct((nc_out, NPP, 128), jnp.float32),
  )


@functools.cache
def _tc_final_kernel():
  def body(acc_ref, g_ref, d_ref, b_ref, out_ref):
    d16p = d_ref[...]
    pk = d16p * (acc_ref[...] + g_ref[...])
    cols = []
    for j in range(8):
      sl = slice(16 * j, 16 * (j + 1))
      p_j = jnp.concatenate([pk[k, :, sl] for k in range(2)], axis=-1)
      res_j = p_j + b_ref[...]
      cols.append(res_j.T[:21, :])
    out_ref[...] = jnp.concatenate(cols, axis=1)

  return pl.pallas_call(
      body,
      grid=(NBLK,),
      in_specs=[
          pl.BlockSpec((2, RBP, 128), lambda i: (0, i, 0)),
          pl.BlockSpec((2, RBP, 128), lambda i: (0, i, 0)),
          pl.BlockSpec((RBP, 128), lambda i: (i, 0)),
          pl.BlockSpec((1, 32), lambda i: (0, 0)),
      ],
      out_specs=pl.BlockSpec((21, RB), lambda i: (0, i)),
      out_shape=jax.ShapeDtypeStruct((21, N), jnp.float32),
  )


def kernel(x, edge_index, W1, b1, W2, b2, W3, b3, W4, b4):
  f32 = jnp.float32
  src = edge_index[0]
  dst = edge_index[1]
  pad_idx = N + (jnp.arange(E_PAD - E, dtype=jnp.int32) % WB)
  src2d = jnp.concatenate([src, pad_idx]).reshape(G_TOTAL, WB)
  dst1d_nat = jnp.concatenate([dst, pad_idx])
  dst2d = dst1d_nat.reshape(G_TOTAL, WB)

  def vperm(n):
    return (n & ~1023) | ((n & 127) << 3) | ((n >> 7) & 7)

  src2v = vperm(src2d).reshape(E_PAD)
  dst2v = vperm(dst2d).reshape(E_PAD)
  zeros1d = jnp.zeros((NP,), f32)
  zeros2d = jnp.zeros((NP, 16), f32)
  zeros3 = jnp.zeros((KG * WB, 16), f32)
  ones_w = jnp.ones((KBD,), f32)

  xt = x.T
  W1p = jnp.zeros((32, 32), f32).at[:21, :].set(W1)
  b1p = b1.reshape(1, 32)
  b2p = b2.reshape(1, 64)
  b3p = b3.reshape(1, 128)
  W4p = jnp.zeros((128, 32), f32).at[:, :21].set(W4)
  b4p = jnp.zeros((1, 32), f32).at[0, :21].set(b4)

  dp = _deg_count_kernel()(dst1d_nat, zeros1d, ones_w)
  dp4 = dp.reshape(2, NBLK, 8, 128)
  d, g1 = _tc0_kernel()(dp4, xt)

  ep2 = _edge_pass_kernel(2)
  ep4 = _edge_pass_kernel(4)

  def sc_view(a):
    return a.reshape(a.shape[0], NP, 16)

  def tc_view(a):
    return a.reshape(a.shape[0], NPP, 128)

  acc1 = tc_view(ep2(src2v, dst2v, sc_view(g1), zeros2d, zeros3))
  g2 = _tc_join_kernel(2, 2, 32, 32, False)(acc1, g1, d, W1p, b1p)
  acc2 = tc_view(ep2(src2v, dst2v, sc_view(g2), zeros2d, zeros3))
  g3 = _tc_join_kernel(2, 4, 32, 64, False)(acc2, g2, d, W2, b2p)
  acc3 = tc_view(ep4(src2v, dst2v, sc_view(g3), zeros2d, zeros3))
  g4 = _tc_join_kernel(4, 2, 64, 128, True)(acc3, g3, d, W3, b3p, W4p)
  acc4 = tc_view(ep2(src2v, dst2v, sc_view(g4), zeros2d, zeros3))
  outt = _tc_final_kernel()(acc4, g4, d, b4p)
  return outt.T


# speedup vs baseline: 1.1081x; 1.0006x over previous
import functools

import jax
import jax.numpy as jnp
from jax import lax
from jax.experimental import pallas as pl
from jax.experimental.pallas import tpu as pltpu
from jax.experimental.pallas import tpu_sc as plsc

N = 100000
E = 3200000
NP = 100352
NC, NS = 2, 16
WB = 128
KG = 4
G_TOTAL = E // WB + 88
E_PAD = G_TOTAL * WB
ROWS_PER_TILE = NP // NS
RB = 1024
NBLK = NP // RB
NPP = NP // 8
RBP = RB // 8

_mesh = lambda: plsc.VectorSubcoreMesh(core_axis_name="c", subcore_axis_name="s")


KGD = 28
KBD = KGD * WB


@functools.cache
def _deg_count_kernel():
  @functools.partial(
      pl.kernel,
      out_type=jax.ShapeDtypeStruct((NC, NP), jnp.float32),
      mesh=_mesh(),
      compiler_params=pltpu.CompilerParams(use_tc_tiling_on_sc=False),
      scratch_types=[
          pltpu.VMEM_SHARED((NP,), jnp.float32),
          pltpu.VMEM((2, KBD), jnp.int32),
          pltpu.VMEM((KBD,), jnp.float32),
          pltpu.SemaphoreType.DMA,
          pltpu.SemaphoreType.DMA,
          pltpu.SemaphoreType.DMA,
      ],
  )
  def deg_count(dst1d, zeros1d, ones_w, out, acc, idx_d, ones_v,
                s0sem, s1sem, isem):
    c = lax.axis_index("c")
    s = lax.axis_index("s")
    ssems = (s0sem, s1sem)
    r0 = s * ROWS_PER_TILE
    pltpu.sync_copy(zeros1d.at[pl.ds(r0, ROWS_PER_TILE)],
                    acc.at[pl.ds(r0, ROWS_PER_TILE)])
    pltpu.sync_copy(ones_w, ones_v)
    plsc.subcore_barrier()
    e_sc = E_PAD // NC
    e_tile = e_sc // NS
    n_chunk = e_tile // KBD
    base = c * e_sc + s * e_tile

    def load_idx(t, buf, sync):
      e0 = base + t * KBD
      if sync:
        pltpu.sync_copy(dst1d.at[pl.ds(e0, KBD)], idx_d.at[buf])
      else:
        pltpu.async_copy(dst1d.at[pl.ds(e0, KBD)], idx_d.at[buf], isem)

    def step(t, buf, first, last):
      if not first:
        pltpu.make_async_copy(zeros1d.at[pl.ds(0, KBD)],
                              idx_d.at[buf], isem).wait()
      pltpu.async_copy(ones_v, acc.at[idx_d.at[buf]], ssems[buf], add=True)
      if not first:
        pltpu.make_async_copy(zeros1d.at[pl.ds(0, KBD)],
                              ones_v, ssems[1 - buf]).wait()
      if not last:
        load_idx(t + 1, 1 - buf, sync=False)

    load_idx(0, 0, sync=True)
    step(0, 0, True, False)

    def body(t2, carry):
      t = 1 + t2 * 2
      step(t, 1, False, False)
      step(t + 1, 0, False, False)
      return carry

    lax.fori_loop(0, (n_chunk - 2) // 2, body, 0)
    step(n_chunk - 1, 1, False, True)
    pltpu.make_async_copy(zeros1d.at[pl.ds(0, KBD)],
                          ones_v, ssems[1]).wait()
    plsc.subcore_barrier()
    pltpu.sync_copy(acc.at[pl.ds(r0, ROWS_PER_TILE)],
                    out.at[c].at[pl.ds(r0, ROWS_PER_TILE)])

  return deg_count


@functools.cache
def _edge_pass_kernel(n_chunks):
  n_per_sc = n_chunks // NC
  g_tile = G_TOTAL // NS
  n_chunk = g_tile // KG

  @functools.partial(
      pl.kernel,
      out_type=jax.ShapeDtypeStruct((n_chunks, NP, 16), jnp.float32),
      mesh=_mesh(),
      compiler_params=pltpu.CompilerParams(use_tc_tiling_on_sc=False),
      scratch_types=[
          pltpu.VMEM_SHARED((NP, 16), jnp.float32),
          pltpu.VMEM((3, KG * WB), jnp.int32),
          pltpu.VMEM((3, KG * WB), jnp.int32),
          pltpu.VMEM((3, KG * WB, 16), jnp.float32),
          pltpu.SemaphoreType.DMA,
          pltpu.SemaphoreType.DMA,
          pltpu.SemaphoreType.DMA,
          pltpu.SemaphoreType.DMA,
          pltpu.SemaphoreType.DMA,
          pltpu.SemaphoreType.DMA,
      ],
  )
  def edge_pass(src1d, dst1d, table, zeros2d, zeros3, out,
                acc, idx_s, idx_d, rows, g0sem, g1sem, g2sem, ssem, isem,
                isem2):
    c = lax.axis_index("c")
    s = lax.axis_index("s")
    gsems = (g0sem, g1sem, g2sem)
    r0 = s * ROWS_PER_TILE
    base = s * g_tile * WB
    kb = KG * WB

    def load_idx(t, buf, sync):
      e0 = base + t * kb
      if sync:
        pltpu.sync_copy(src1d.at[pl.ds(e0, kb)], idx_s.at[buf])
        pltpu.sync_copy(dst1d.at[pl.ds(e0, kb)], idx_d.at[buf])
      else:
        pltpu.async_copy(src1d.at[pl.ds(e0, kb)], idx_s.at[buf], isem)
        pltpu.async_copy(dst1d.at[pl.ds(e0, kb)], idx_d.at[buf], isem2)

    def fire_gathers(chunk, buf):
      h = kb // 2
      pltpu.async_copy(table.at[chunk].at[idx_s.at[buf].at[pl.ds(0, h)]],
                       rows.at[buf].at[pl.ds(0, h)], gsems[buf])
      pltpu.async_copy(table.at[chunk].at[idx_s.at[buf].at[pl.ds(h, h)]],
                       rows.at[buf].at[pl.ds(h, h)], gsems[buf])

    def drain_rows(sem, buf):
      pltpu.make_async_copy(zeros3, rows.at[buf], sem).wait()

    def drain_idx():
      pltpu.make_async_copy(src1d.at[pl.ds(0, kb)], idx_s.at[0], isem).wait()
      pltpu.make_async_copy(src1d.at[pl.ds(0, kb)], idx_d.at[0], isem2).wait()

    def step(chunk, t, a, fire_g, fire_i):
      drain_rows(gsems[a], a)
      pltpu.async_copy(rows.at[a], acc.at[idx_d.at[a]], ssem, add=True)
      if fire_g:
        drain_idx()
        fire_gathers(chunk, (a + 2) % 3)
      drain_rows(ssem, a)
      if fire_i:
        load_idx(t + 3, a, sync=False)

    for ci in range(n_per_sc):
      chunk = c + NC * ci
      pltpu.sync_copy(zeros2d.at[pl.ds(r0, ROWS_PER_TILE)],
                      acc.at[pl.ds(r0, ROWS_PER_TILE)])
      plsc.subcore_barrier()

      load_idx(0, 0, sync=True)
      load_idx(1, 1, sync=True)
      fire_gathers(chunk, 0)
      fire_gathers(chunk, 1)
      load_idx(2, 2, sync=False)

      def body(t2, carry):
        t = t2 * 3
        step(chunk, t, 0, True, True)
        step(chunk, t + 1, 1, True, True)
        step(chunk, t + 2, 2, True, True)
        return carry

      n_steady = n_chunk - 5
      lax.fori_loop(0, n_steady // 3, body, 0)
      step(chunk, n_chunk - 5, 0, True, True)
      step(chunk, n_chunk - 4, 1, True, True)
      step(chunk, n_chunk - 3, 2, True, False)
      step(chunk, n_chunk - 2, 0, False, False)
      step(chunk, n_chunk - 1, 1, False, False)

      plsc.subcore_barrier()
      pltpu.sync_copy(acc.at[pl.ds(r0, ROWS_PER_TILE)],
                      out.at[chunk].at[pl.ds(r0, ROWS_PER_TILE)])
      if ci + 1 < n_per_sc:
        plsc.subcore_barrier()

  return edge_pass


def _row_mask(val):
  i = pl.program_id(0)
  rows = i * RB + lax.broadcasted_iota(jnp.int32, (RB, 1), 0)
  return jnp.where(rows < N, val, 0.0)


def _chunk_cat(ref, n):
  return jnp.concatenate([ref[k] for k in range(n)], axis=-1)


def _jmask(j, val):
  i = pl.program_id(0)
  rows = i * RB + j * 128 + lax.broadcasted_iota(jnp.int32, (128, 1), 0)
  return jnp.where(rows < N, val, 0.0)


@functools.cache
def _tc0_kernel():
  def body(dp_ref, xt_ref, d16_ref, g_ref):
    deg = 1.0 + dp_ref[0, 0] + dp_ref[1, 0]
    dd8 = lax.rsqrt(deg).T
    d16_cols = []
    g_cols = [[], []]
    for j in range(8):
      d16_j = jnp.broadcast_to(dd8[:, j:j + 1], (128, 16))
      d16_cols.append(d16_j)
      xj = xt_ref[:, 128 * j:128 * (j + 1)].T
      x32 = jnp.concatenate([xj, jnp.zeros((128, 11), jnp.float32)], axis=1)
      gj = _jmask(j, jnp.concatenate([d16_j, d16_j], axis=1) * x32)
      g_cols[0].append(gj[:, :16])
      g_cols[1].append(gj[:, 16:])
    d16_ref[...] = jnp.concatenate(d16_cols, axis=1)
    g_ref[0] = jnp.concatenate(g_cols[0], axis=1)
    g_ref[1] = jnp.concatenate(g_cols[1], axis=1)

  return pl.pallas_call(
      body,
      grid=(NBLK,),
      in_specs=[
          pl.BlockSpec((2, 1, 8, 128), lambda i: (0, i, 0, 0)),
          pl.BlockSpec((21, RB), lambda i: (0, i)),
      ],
      out_specs=[
          pl.BlockSpec((RBP, 128), lambda i: (i, 0)),
          pl.BlockSpec((2, RBP, 128), lambda i: (0, i, 0)),
      ],
      out_shape=[
          jax.ShapeDtypeStruct((NPP, 128), jnp.float32),
          jax.ShapeDtypeStruct((2, NPP, 128), jnp.float32),
      ],
  )


@functools.cache
def _tc_join_kernel(nc_in, nc_out, cin, cout, two_mats):
  def body(acc_ref, g_ref, d_ref, w_ref, b_ref, *rest):
    if two_mats:
      w2_ref, out_ref = rest
    else:
      (out_ref,) = rest
    d16p = d_ref[...]
    pk = d16p * (acc_ref[...] + g_ref[...])
    p = jnp.concatenate(
        [jnp.concatenate([pk[k, :, 16 * j:16 * (j + 1)] for k in range(nc_in)],
                         axis=-1) for j in range(8)], axis=0)
    x = jnp.maximum(
        jnp.dot(p, w_ref[...], preferred_element_type=jnp.float32)
        + b_ref[...], 0.0)
    if two_mats:
      x = jnp.dot(x, w2_ref[...], preferred_element_type=jnp.float32)
    x = _row_mask(x)
    for m in range(nc_out):
      out_ref[m] = d16p * jnp.concatenate(
          [x[128 * j:128 * (j + 1), 16 * m:16 * (m + 1)] for j in range(8)],
          axis=1)

  in_specs = [
      pl.BlockSpec((nc_in, RBP, 128), lambda i: (0, i, 0)),
      pl.BlockSpec((nc_in, RBP, 128), lambda i: (0, i, 0)),
      pl.BlockSpec((RBP, 128), lambda i: (i, 0)),
      pl.BlockSpec((cin, cout), lambda i: (0, 0)),
      pl.BlockSpec((1, cout), lambda i: (0, 0)),
  ]
  if two_mats:
    in_specs.append(pl.BlockSpec((cout, 16 * nc_out), lambda i: (0, 0)))
  return pl.pallas_call(
      body,
      grid=(NBLK,),
      in_specs=in_specs,
      out_specs=pl.BlockSpec((nc_out, RBP, 128), lambda i: (0, i, 0)),
      out_shape=jax.ShapeDtypeStruct((nc_out, NPP, 128), jnp.float32),
  )


@functools.cache
def _tc_final_kernel():
  def body(acc_ref, g_ref, d_ref, b_ref, out_ref):
    d16p = d_ref[...]
    pk = d16p * (acc_ref[...] + g_ref[...])
    cols = []
    for j in range(8):
      sl = slice(16 * j, 16 * (j + 1))
      p_j = jnp.concatenate([pk[k, :, sl] for k in range(2)], axis=-1)
      res_j = p_j + b_ref[...]
      cols.append(res_j.T[:21, :])
    out_ref[...] = jnp.concatenate(cols, axis=1)

  return pl.pallas_call(
      body,
      grid=(NBLK,),
      in_specs=[
          pl.BlockSpec((2, RBP, 128), lambda i: (0, i, 0)),
          pl.BlockSpec((2, RBP, 128), lambda i: (0, i, 0)),
          pl.BlockSpec((RBP, 128), lambda i: (i, 0)),
          pl.BlockSpec((1, 32), lambda i: (0, 0)),
      ],
      out_specs=pl.BlockSpec((21, RB), lambda i: (0, i)),
      out_shape=jax.ShapeDtypeStruct((21, N), jnp.float32),
  )


def kernel(x, edge_index, W1, b1, W2, b2, W3, b3, W4, b4):
  f32 = jnp.float32
  src = edge_index[0]
  dst = edge_index[1]
  pad_idx = N + (jnp.arange(E_PAD - E, dtype=jnp.int32) % WB)
  src2d = jnp.concatenate([src, pad_idx]).reshape(G_TOTAL, WB)
  dst1d_nat = jnp.concatenate([dst, pad_idx])
  dst2d = dst1d_nat.reshape(G_TOTAL, WB)

  def vperm(n):
    return (n & ~1023) | ((n & 127) << 3) | ((n >> 7) & 7)

  src2v = vperm(src2d).reshape(E_PAD)
  dst2v = vperm(dst2d).reshape(E_PAD)
  zeros1d = jnp.zeros((NP,), f32)
  zeros2d = jnp.zeros((NP, 16), f32)
  zeros3 = jnp.zeros((KG * WB, 16), f32)
  ones_w = jnp.ones((KBD,), f32)

  xt = x.T
  W1p = jnp.zeros((32, 32), f32).at[:21, :].set(W1)
  b1p = b1.reshape(1, 32)
  b2p = b2.reshape(1, 64)
  b3p = b3.reshape(1, 128)
  W4p = jnp.zeros((128, 32), f32).at[:, :21].set(W4)
  b4p = jnp.zeros((1, 32), f32).at[0, :21].set(b4)

  dp = _deg_count_kernel()(dst1d_nat, zeros1d, ones_w)
  dp4 = dp.reshape(2, NBLK, 8, 128)
  d, g1 = _tc0_kernel()(dp4, xt)

  ep2 = _edge_pass_kernel(2)
  ep4 = _edge_pass_kernel(4)

  def sc_view(a):
    return a.reshape(a.shape[0], NP, 16)

  def tc_view(a):
    return a.reshape(a.shape[0], NPP, 128)

  acc1 = tc_view(ep2(src2v, dst2v, sc_view(g1), zeros2d, zeros3))
  g2 = _tc_join_kernel(2, 2, 32, 32, False)(acc1, g1, d, W1p, b1p)
  acc2 = tc_view(ep2(src2v, dst2v, sc_view(g2), zeros2d, zeros3))
  g3 = _tc_join_kernel(2, 4, 32, 64, False)(acc2, g2, d, W2, b2p)
  acc3 = tc_view(ep4(src2v, dst2v, sc_view(g3), zeros2d, zeros3))
  g4 = _tc_join_kernel(4, 2, 64, 128, True)(acc3, g3, d, W3, b3p, W4p)
  acc4 = tc_view(ep2(src2v, dst2v, sc_view(g4), zeros2d, zeros3))
  outt = _tc_final_kernel()(acc4, g4, d, b4p)
  return outt.T
